# Initial kernel scaffold; baseline (speedup 1.0000x reference)
import jax
import jax.numpy as jnp
from jax.experimental import pallas as pl


def kernel(X, realid, weight_fixed, tuned_weight, tuned_vector, W_lin):
    raise NotImplementedError("write your pallas kernel here")


# speedup vs baseline: 1.8421x; 1.8421x over previous
import functools

import jax
import jax.numpy as jnp
from jax import lax
from jax.experimental import pallas as pl
from jax.experimental.pallas import tpu as pltpu
from jax.experimental.pallas import tpu_sc as plsc

_NWORD = 100000
_N_FIXED = 80000
_VEC = 300
_ADD = 212
_IN_DIM = _VEC + _ADD
_OUT = 512
_BATCH, _SEQ = 4096, 50
_B = _BATCH * _SEQ


_ROWS = 1000
_NF_BLOCKS = _N_FIXED // _ROWS
_N_BLOCKS = _NWORD // _ROWS


def _project_body(wf_ref, tw_ref, tv_ref, wl_ref, out_ref):
    i = pl.program_id(0)
    vec = jnp.where(i < _NF_BLOCKS, wf_ref[...], tw_ref[...])
    wl = wl_ref[...]
    p = lax.dot_general(vec, wl[:, :_VEC], (((1,), (1,)), ((), ())),
                        preferred_element_type=jnp.float32)
    p = p + lax.dot_general(tv_ref[...], wl[:, _VEC:], (((1,), (1,)), ((), ())),
                            preferred_element_type=jnp.float32)
    out_ref[...] = p


def _project_table(weight_fixed, tuned_weight, tuned_vector, W_lin):
    return pl.pallas_call(
        _project_body,
        grid=(_N_BLOCKS,),
        in_specs=[
            pl.BlockSpec((_ROWS, _VEC),
                         lambda i: (jnp.minimum(i, _NF_BLOCKS - 1), 0)),
            pl.BlockSpec((_ROWS, _VEC),
                         lambda i: (jnp.maximum(i - _NF_BLOCKS, 0), 0)),
            pl.BlockSpec((_ROWS, _ADD), lambda i: (i, 0)),
            pl.BlockSpec((_OUT, _IN_DIM), lambda i: (0, 0)),
        ],
        out_specs=pl.BlockSpec((_ROWS, _OUT), lambda i: (i, 0)),
        out_shape=jax.ShapeDtypeStruct((_NWORD, _OUT), jnp.float32),
    )(weight_fixed, tuned_weight, tuned_vector, W_lin)


_NC, _NS = 2, 16
_NW = _NC * _NS
_B_PER_W = _B // _NW
_CHUNK = 64
_N_CHUNKS = _B_PER_W // _CHUNK


def _gather_body(x_hbm, realid_hbm, table_hbm, out_hbm, x_v, xm_v, rows_v, sem):
    wid = lax.axis_index("s") * _NC + lax.axis_index("c")
    base = wid * _B_PER_W
    pltpu.sync_copy(x_hbm.at[pl.ds(base, _B_PER_W)], x_v)
    pltpu.async_copy(realid_hbm.at[x_v], xm_v, sem).wait()

    def body(c, carry):
        pltpu.async_copy(
            table_hbm.at[xm_v.at[pl.ds(c * _CHUNK, _CHUNK)]], rows_v, sem
        ).wait()
        pltpu.sync_copy(rows_v, out_hbm.at[pl.ds(base + c * _CHUNK, _CHUNK)])
        return carry

    lax.fori_loop(0, _N_CHUNKS, body, 0)


_gather_rows = functools.partial(
    pl.kernel,
    mesh=plsc.VectorSubcoreMesh(core_axis_name="c", subcore_axis_name="s"),
    out_type=jax.ShapeDtypeStruct((_B, _OUT), jnp.float32),
    scratch_types=[
        pltpu.VMEM((_B_PER_W,), jnp.int32),
        pltpu.VMEM((_B_PER_W,), jnp.int32),
        pltpu.VMEM((_CHUNK, _OUT), jnp.float32),
        pltpu.SemaphoreType.DMA,
    ],
)(_gather_body)


def kernel(X, realid, weight_fixed, tuned_weight, tuned_vector, W_lin):
    table = _project_table(weight_fixed, tuned_weight, tuned_vector, W_lin)
    xflat = X.reshape(-1).astype(jnp.int32)
    rid = realid.astype(jnp.int32)
    out = _gather_rows(xflat, rid, table)
    return out.reshape(_BATCH, _SEQ, _OUT)


# speedup vs baseline: 2.8383x; 1.5408x over previous
import functools

import jax
import jax.numpy as jnp
from jax import lax
from jax.experimental import pallas as pl
from jax.experimental.pallas import tpu as pltpu
from jax.experimental.pallas import tpu_sc as plsc

_NWORD = 100000
_N_FIXED = 80000
_VEC = 300
_ADD = 212
_IN_DIM = _VEC + _ADD
_OUT = 512
_BATCH, _SEQ = 4096, 50
_B = _BATCH * _SEQ


_ROWS = 1000
_NF_BLOCKS = _N_FIXED // _ROWS
_N_BLOCKS = _NWORD // _ROWS


def _project_body(wf_ref, tw_ref, tv_ref, wl_ref, out_ref):
    i = pl.program_id(0)
    vec = jnp.where(i < _NF_BLOCKS, wf_ref[...], tw_ref[...])
    vec = vec.astype(jnp.bfloat16)
    wl = wl_ref[...].astype(jnp.bfloat16)
    p = lax.dot_general(vec, wl[:, :_VEC], (((1,), (1,)), ((), ())),
                        preferred_element_type=jnp.float32)
    p = p + lax.dot_general(tv_ref[...].astype(jnp.bfloat16), wl[:, _VEC:],
                            (((1,), (1,)), ((), ())),
                            preferred_element_type=jnp.float32)
    out_ref[...] = p.reshape(_ROWS * _OUT)


def _project_table(weight_fixed, tuned_weight, tuned_vector, W_lin):
    return pl.pallas_call(
        _project_body,
        grid=(_N_BLOCKS,),
        in_specs=[
            pl.BlockSpec((_ROWS, _VEC),
                         lambda i: (jnp.minimum(i, _NF_BLOCKS - 1), 0)),
            pl.BlockSpec((_ROWS, _VEC),
                         lambda i: (jnp.maximum(i - _NF_BLOCKS, 0), 0)),
            pl.BlockSpec((_ROWS, _ADD), lambda i: (i, 0)),
            pl.BlockSpec((_OUT, _IN_DIM), lambda i: (0, 0)),
        ],
        out_specs=pl.BlockSpec((_ROWS * _OUT,), lambda i: (i,)),
        out_shape=jax.ShapeDtypeStruct((_NWORD * _OUT,), jnp.float32),
    )(weight_fixed, tuned_weight, tuned_vector, W_lin)


_NC, _NS = 2, 16
_NW = _NC * _NS
_B_PER_W = _B // _NW
_CHUNK = 40
_NBUF = 4
_N_CHUNKS = _B_PER_W // _CHUNK
_N_ITERS = _N_CHUNKS // _NBUF


def _gather_body(x_hbm, realid_hbm, table_hbm, out_hbm,
                 t_v, x_v, xm_v, r0, r1, r2, r3,
                 g0, g1, g2, g3, w0, w1, w2, w3):
    rows = (r0, r1, r2, r3)
    gsem = (g0, g1, g2, g3)
    wsem = (w0, w1, w2, w3)
    wid = lax.axis_index("s") * _NC + lax.axis_index("c")
    base = wid * _B_PER_W

    def idx_body(j, carry):
        u = base + j * 16 + lax.iota(jnp.int32, 16)
        t = (u & (_BATCH - 1)) * _SEQ + (u >> 12)
        t_v[pl.ds(j * 16, 16)] = t
        return carry

    lax.fori_loop(0, _B_PER_W // 16, idx_body, 0)
    pltpu.async_copy(x_hbm.at[t_v], x_v, g0).wait()
    pltpu.async_copy(realid_hbm.at[x_v], xm_v, g0).wait()

    def _g_start(c, b):
        pltpu.async_copy(
            table_hbm.at[xm_v.at[pl.ds(c * _CHUNK, _CHUNK)]], rows[b], gsem[b])

    def _g_wait(b):
        pltpu.make_async_copy(
            table_hbm.at[xm_v.at[pl.ds(0, _CHUNK)]], rows[b], gsem[b]).wait()

    def _w_start(c, b):
        pltpu.async_copy(
            rows[b], out_hbm.at[pl.ds(base + c * _CHUNK, _CHUNK)], wsem[b])

    def _w_wait(b):
        pltpu.make_async_copy(
            rows[b], out_hbm.at[pl.ds(base, _CHUNK)], wsem[b]).wait()

    for b in range(_NBUF):
        _g_start(b, b)

    def body(g, carry):
        for b in range(_NBUF):
            c = g * _NBUF + b
            _g_wait(b)
            _w_start(c, b)
            nc = c + _NBUF

            @pl.when(nc < _N_CHUNKS)
            def _():
                _w_wait(b)
                _g_start(nc, b)
        return carry

    lax.fori_loop(0, _N_ITERS, body, 0)
    for b in range(_NBUF):
        _w_wait(b)


_gather_rows = functools.partial(
    pl.kernel,
    mesh=plsc.VectorSubcoreMesh(core_axis_name="c", subcore_axis_name="s"),
    out_type=jax.ShapeDtypeStruct((_B, _OUT), jnp.float32),
    scratch_types=(
        [pltpu.VMEM((_B_PER_W,), jnp.int32)] * 3
        + [pltpu.VMEM((_CHUNK, _OUT), jnp.float32)] * _NBUF
        + [pltpu.SemaphoreType.DMA] * (2 * _NBUF)
    ),
)(_gather_body)


def kernel(X, realid, weight_fixed, tuned_weight, tuned_vector, W_lin):
    table = _project_table(weight_fixed, tuned_weight, tuned_vector, W_lin)
    table = table.reshape(_NWORD, _OUT)
    xflat = X.reshape(-1).astype(jnp.int32)
    rid = realid.astype(jnp.int32)
    out = _gather_rows(xflat, rid, table)
    return out.reshape(_SEQ, _BATCH, _OUT).transpose(1, 0, 2)


# speedup vs baseline: 3.6491x; 1.2857x over previous
import functools

import jax
import jax.numpy as jnp
from jax import lax
from jax.experimental import pallas as pl
from jax.experimental.pallas import tpu as pltpu
from jax.experimental.pallas import tpu_sc as plsc

_NWORD = 100000
_N_FIXED = 80000
_VEC = 300
_ADD = 212
_IN_DIM = _VEC + _ADD
_OUT = 512
_BATCH, _SEQ = 4096, 50
_B = _BATCH * _SEQ


_ROWS = 1000
_NF_BLOCKS = _N_FIXED // _ROWS
_N_BLOCKS = _NWORD // _ROWS


def _project_body(wf_ref, tw_ref, tv_ref, wl_ref, out_ref):
    i = pl.program_id(0)
    vec = jnp.where(i < _NF_BLOCKS, wf_ref[...], tw_ref[...])
    vec = vec.astype(jnp.bfloat16)
    wl = wl_ref[...].astype(jnp.bfloat16)
    p = lax.dot_general(vec, wl[:, :_VEC], (((1,), (1,)), ((), ())),
                        preferred_element_type=jnp.float32)
    p = p + lax.dot_general(tv_ref[...].astype(jnp.bfloat16), wl[:, _VEC:],
                            (((1,), (1,)), ((), ())),
                            preferred_element_type=jnp.float32)
    out_ref[...] = p


def _project_table(weight_fixed, tuned_weight, tuned_vector, W_lin):
    return pl.pallas_call(
        _project_body,
        grid=(_N_BLOCKS,),
        in_specs=[
            pl.BlockSpec((_ROWS, _VEC),
                         lambda i: (jnp.minimum(i, _NF_BLOCKS - 1), 0)),
            pl.BlockSpec((_ROWS, _VEC),
                         lambda i: (jnp.maximum(i - _NF_BLOCKS, 0), 0)),
            pl.BlockSpec((_ROWS, _ADD), lambda i: (i, 0)),
            pl.BlockSpec((_OUT, _IN_DIM), lambda i: (0, 0)),
        ],
        out_specs=pl.BlockSpec((_ROWS, _OUT), lambda i: (i, 0)),
        out_shape=jax.ShapeDtypeStruct((_NWORD, _OUT), jnp.float32),
    )(weight_fixed, tuned_weight, tuned_vector, W_lin)


_NC, _NS = 2, 16
_NW = _NC * _NS
_B_PER_W = _B // _NW
_CHUNK = 40
_NBUF = 4
_N_CHUNKS = _B_PER_W // _CHUNK
_N_ITERS = _N_CHUNKS // _NBUF


def _gather_body(x_hbm, realid_hbm, table_hbm, out_hbm,
                 x_v, xm_v, r0, r1, r2, r3,
                 g0, g1, g2, g3, w0, w1, w2, w3):
    rows = (r0, r1, r2, r3)
    gsem = (g0, g1, g2, g3)
    wsem = (w0, w1, w2, w3)
    wid = lax.axis_index("s") * _NC + lax.axis_index("c")
    base = wid * _B_PER_W
    pltpu.sync_copy(x_hbm.at[pl.ds(base, _B_PER_W)], x_v)
    pltpu.async_copy(realid_hbm.at[x_v], xm_v, g0).wait()

    def _g_start(c, b):
        pltpu.async_copy(
            table_hbm.at[xm_v.at[pl.ds(c * _CHUNK, _CHUNK)]], rows[b], gsem[b])

    def _g_wait(b):
        pltpu.make_async_copy(
            table_hbm.at[xm_v.at[pl.ds(0, _CHUNK)]], rows[b], gsem[b]).wait()

    def _w_start(c, b):
        pltpu.async_copy(
            rows[b], out_hbm.at[pl.ds(base + c * _CHUNK, _CHUNK)], wsem[b])

    def _w_wait(b):
        pltpu.make_async_copy(
            rows[b], out_hbm.at[pl.ds(base, _CHUNK)], wsem[b]).wait()

    for b in range(_NBUF):
        _g_start(b, b)

    def body(g, carry):
        for b in range(_NBUF):
            c = g * _NBUF + b
            _g_wait(b)
            _w_start(c, b)
            nc = c + _NBUF

            @pl.when(nc < _N_CHUNKS)
            def _():
                _w_wait(b)
                _g_start(nc, b)
        return carry

    lax.fori_loop(0, _N_ITERS, body, 0)
    for b in range(_NBUF):
        _w_wait(b)


_gather_rows = functools.partial(
    pl.kernel,
    mesh=plsc.VectorSubcoreMesh(core_axis_name="c", subcore_axis_name="s"),
    out_type=jax.ShapeDtypeStruct((_B, _OUT), jnp.float32),
    scratch_types=(
        [pltpu.VMEM((_B_PER_W,), jnp.int32)] * 2
        + [pltpu.VMEM((_CHUNK, _OUT), jnp.float32)] * _NBUF
        + [pltpu.SemaphoreType.DMA] * (2 * _NBUF)
    ),
)(_gather_body)


def kernel(X, realid, weight_fixed, tuned_weight, tuned_vector, W_lin):
    table = _project_table(weight_fixed, tuned_weight, tuned_vector, W_lin)
    xflat = X.T.reshape(-1).astype(jnp.int32)
    rid = realid.astype(jnp.int32)
    out = _gather_rows(xflat, rid, table)
    return out.reshape(_SEQ, _BATCH, _OUT).transpose(1, 0, 2)


# speedup vs baseline: 4.9944x; 1.3687x over previous
import functools

import jax
import jax.numpy as jnp
from jax import lax
from jax.experimental import pallas as pl
from jax.experimental.pallas import tpu as pltpu
from jax.experimental.pallas import tpu_sc as plsc

_NWORD = 100000
_N_FIXED = 80000
_VEC = 300
_ADD = 212
_IN_DIM = _VEC + _ADD
_OUT = 512
_BATCH, _SEQ = 4096, 50
_B = _BATCH * _SEQ


_ROWS = 640
_NF_BLOCKS = _N_FIXED // _ROWS
_N_BLOCKS = (_NWORD + _ROWS - 1) // _ROWS


def _project_body(wf_ref, tw_ref, tv_ref, wl_ref, out_ref):
    i = pl.program_id(0)
    vec = jnp.where(i < _NF_BLOCKS, wf_ref[...], tw_ref[...])
    vec = vec.astype(jnp.bfloat16)
    wl = wl_ref[...].astype(jnp.bfloat16)
    p = lax.dot_general(vec, wl[:, :_VEC], (((0,), (1,)), ((), ())),
                        preferred_element_type=jnp.float32)
    p = p + lax.dot_general(tv_ref[...].astype(jnp.bfloat16), wl[:, _VEC:],
                            (((0,), (1,)), ((), ())),
                            preferred_element_type=jnp.float32)
    out_ref[...] = p


def _project_table(weight_fixed, tuned_weight, tuned_vector, W_lin):
    return pl.pallas_call(
        _project_body,
        grid=(_N_BLOCKS,),
        in_specs=[
            pl.BlockSpec((_VEC, _ROWS),
                         lambda i: (0, jnp.minimum(i, _NF_BLOCKS - 1))),
            pl.BlockSpec((_VEC, _ROWS),
                         lambda i: (0, jnp.maximum(i - _NF_BLOCKS, 0))),
            pl.BlockSpec((_ADD, _ROWS), lambda i: (0, i)),
            pl.BlockSpec((_OUT, _IN_DIM), lambda i: (0, 0)),
        ],
        out_specs=pl.BlockSpec((_ROWS, _OUT), lambda i: (i, 0)),
        out_shape=jax.ShapeDtypeStruct((_NWORD, _OUT), jnp.float32),
    )(weight_fixed.T, tuned_weight.T, tuned_vector.T, W_lin)


_NC, _NS = 2, 16
_NW = _NC * _NS
_B_PER_W = _B // _NW
_CHUNK = 40
_NBUF = 4
_N_CHUNKS = _B_PER_W // _CHUNK
_N_ITERS = _N_CHUNKS // _NBUF


def _gather_body(x_hbm, realid_hbm, table_hbm, out_hbm,
                 x_v, xm_v, r0, r1, r2, r3,
                 g0, g1, g2, g3, w0, w1, w2, w3):
    rows = (r0, r1, r2, r3)
    gsem = (g0, g1, g2, g3)
    wsem = (w0, w1, w2, w3)
    wid = lax.axis_index("s") * _NC + lax.axis_index("c")
    base = wid * _B_PER_W
    pltpu.sync_copy(x_hbm.at[pl.ds(base, _B_PER_W)], x_v)
    pltpu.async_copy(realid_hbm.at[x_v], xm_v, g0).wait()

    def _g_start(c, b):
        pltpu.async_copy(
            table_hbm.at[xm_v.at[pl.ds(c * _CHUNK, _CHUNK)]], rows[b], gsem[b])

    def _g_wait(b):
        pltpu.make_async_copy(
            table_hbm.at[xm_v.at[pl.ds(0, _CHUNK)]], rows[b], gsem[b]).wait()

    def _w_start(c, b):
        pltpu.async_copy(
            rows[b], out_hbm.at[pl.ds(base + c * _CHUNK, _CHUNK)], wsem[b])

    def _w_wait(b):
        pltpu.make_async_copy(
            rows[b], out_hbm.at[pl.ds(base, _CHUNK)], wsem[b]).wait()

    for b in range(_NBUF):
        _g_start(b, b)

    def body(g, carry):
        for b in range(_NBUF):
            c = g * _NBUF + b
            _g_wait(b)
            _w_start(c, b)
            nc = c + _NBUF

            @pl.when(nc < _N_CHUNKS)
            def _():
                _w_wait(b)
                _g_start(nc, b)
        return carry

    lax.fori_loop(0, _N_ITERS, body, 0)
    for b in range(_NBUF):
        _w_wait(b)


_gather_rows = functools.partial(
    pl.kernel,
    mesh=plsc.VectorSubcoreMesh(core_axis_name="c", subcore_axis_name="s"),
    out_type=jax.ShapeDtypeStruct((_B, _OUT), jnp.float32),
    scratch_types=(
        [pltpu.VMEM((_B_PER_W,), jnp.int32)] * 2
        + [pltpu.VMEM((_CHUNK, _OUT), jnp.float32)] * _NBUF
        + [pltpu.SemaphoreType.DMA] * (2 * _NBUF)
    ),
)(_gather_body)


def kernel(X, realid, weight_fixed, tuned_weight, tuned_vector, W_lin):
    table = _project_table(weight_fixed, tuned_weight, tuned_vector, W_lin)
    xflat = X.T.reshape(-1).astype(jnp.int32)
    rid = realid.astype(jnp.int32)
    out = _gather_rows(xflat, rid, table)
    return out.reshape(_SEQ, _BATCH, _OUT).transpose(1, 0, 2)


# speedup vs baseline: 5.9475x; 1.1908x over previous
import functools

import jax
import jax.numpy as jnp
from jax import lax
from jax.experimental import pallas as pl
from jax.experimental.pallas import tpu as pltpu
from jax.experimental.pallas import tpu_sc as plsc

_NWORD = 100000
_N_FIXED = 80000
_VEC = 300
_ADD = 212
_IN_DIM = _VEC + _ADD
_OUT = 512
_BATCH, _SEQ = 4096, 50
_B = _BATCH * _SEQ


_ROWS = 3200
_NF_BLOCKS = _N_FIXED // _ROWS
_N_BLOCKS = (_NWORD + _ROWS - 1) // _ROWS


def _project_body(wf_ref, tw_ref, tv_ref, wl_ref, out_ref):
    i = pl.program_id(0)
    vec = jnp.where(i < _NF_BLOCKS, wf_ref[...], tw_ref[...])
    vec = vec.astype(jnp.bfloat16)
    wl = wl_ref[...].astype(jnp.bfloat16)
    p = lax.dot_general(vec, wl[:, :_VEC], (((0,), (1,)), ((), ())),
                        preferred_element_type=jnp.float32)
    p = p + lax.dot_general(tv_ref[...].astype(jnp.bfloat16), wl[:, _VEC:],
                            (((0,), (1,)), ((), ())),
                            preferred_element_type=jnp.float32)
    out_ref[...] = p


def _project_table(weight_fixed, tuned_weight, tuned_vector, W_lin):
    return pl.pallas_call(
        _project_body,
        grid=(_N_BLOCKS,),
        in_specs=[
            pl.BlockSpec((_VEC, _ROWS),
                         lambda i: (0, jnp.minimum(i, _NF_BLOCKS - 1))),
            pl.BlockSpec((_VEC, _ROWS),
                         lambda i: (0, jnp.maximum(i - _NF_BLOCKS, 0))),
            pl.BlockSpec((_ADD, _ROWS), lambda i: (0, i)),
            pl.BlockSpec((_OUT, _IN_DIM), lambda i: (0, 0)),
        ],
        out_specs=pl.BlockSpec((_ROWS, _OUT), lambda i: (i, 0)),
        out_shape=jax.ShapeDtypeStruct((_NWORD, _OUT), jnp.float32),
    )(weight_fixed.T, tuned_weight.T, tuned_vector.T, W_lin)


_NC, _NS = 2, 16
_NW = _NC * _NS
_B_PER_W = _B // _NW
_CHUNK = 40
_NBUF = 5
_N_CHUNKS = _B_PER_W // _CHUNK
_N_ITERS = _N_CHUNKS // _NBUF


def _gather_body(x_hbm, realid_hbm, table_hbm, out_hbm,
                 x_v, xm_v, r0, r1, r2, r3, r4,
                 g0, g1, g2, g3, g4, w0, w1, w2, w3, w4):
    rows = (r0, r1, r2, r3, r4)
    gsem = (g0, g1, g2, g3, g4)
    wsem = (w0, w1, w2, w3, w4)
    wid = lax.axis_index("s") * _NC + lax.axis_index("c")
    base = wid * _B_PER_W
    pltpu.sync_copy(x_hbm.at[pl.ds(base, _B_PER_W)], x_v)
    pltpu.async_copy(realid_hbm.at[x_v], xm_v, g0).wait()

    def _g_start(c, b):
        pltpu.async_copy(
            table_hbm.at[xm_v.at[pl.ds(c * _CHUNK, _CHUNK)]], rows[b], gsem[b])

    def _g_wait(b):
        pltpu.make_async_copy(
            table_hbm.at[xm_v.at[pl.ds(0, _CHUNK)]], rows[b], gsem[b]).wait()

    def _w_start(c, b):
        pltpu.async_copy(
            rows[b], out_hbm.at[pl.ds(base + c * _CHUNK, _CHUNK)], wsem[b])

    def _w_wait(b):
        pltpu.make_async_copy(
            rows[b], out_hbm.at[pl.ds(base, _CHUNK)], wsem[b]).wait()

    for b in range(_NBUF):
        _g_start(b, b)

    def body(g, carry):
        for b in range(_NBUF):
            c = g * _NBUF + b
            _g_wait(b)
            _w_start(c, b)
            nc = c + _NBUF

            @pl.when(nc < _N_CHUNKS)
            def _():
                _w_wait(b)
                _g_start(nc, b)
        return carry

    lax.fori_loop(0, _N_ITERS, body, 0)
    for b in range(_NBUF):
        _w_wait(b)


_gather_rows = functools.partial(
    pl.kernel,
    mesh=plsc.VectorSubcoreMesh(core_axis_name="c", subcore_axis_name="s"),
    out_type=jax.ShapeDtypeStruct((_B, _OUT), jnp.float32),
    scratch_types=(
        [pltpu.VMEM((_B_PER_W,), jnp.int32)] * 2
        + [pltpu.VMEM((_CHUNK, _OUT), jnp.float32)] * _NBUF
        + [pltpu.SemaphoreType.DMA] * (2 * _NBUF)
    ),
)(_gather_body)


def kernel(X, realid, weight_fixed, tuned_weight, tuned_vector, W_lin):
    table = _project_table(weight_fixed, tuned_weight, tuned_vector, W_lin)
    xflat = X.T.reshape(-1).astype(jnp.int32)
    rid = realid.astype(jnp.int32)
    out = _gather_rows(xflat, rid, table)
    return out.reshape(_SEQ, _BATCH, _OUT).transpose(1, 0, 2)
